# baseline (speedup 1.0000x reference)
import functools

import jax
import jax.numpy as jnp
from jax import lax
from jax.experimental import pallas as pl
from jax.experimental.pallas import tpu as pltpu
from jax.experimental.pallas import tpu_sc as plsc

NC = 2
NS = 16
NW = NC * NS
CHUNK = 128
BLK = 1000
LANES = 16
AGG_SPLIT = 1.0


def _sc_deg_body(chunks, rows_per_tile, dst_hbm, out_hbm, d0, d1, d2, d3,
                 ones_v, zrow_v, isem, ssem0, ssem1, shared_deg):
  cid = lax.axis_index("c")
  sid = lax.axis_index("s")
  wid = cid * NS + sid
  base = wid * chunks * CHUNK
  didxs = (d0, d1, d2, d3)
  ssems = (ssem0, ssem1)

  def initc(i, _):
    ones_v[pl.ds(i * LANES, LANES)] = jnp.ones((LANES,), jnp.float32)
    return _

  lax.fori_loop(0, CHUNK // LANES, initc, None)

  def initz(i, _):
    zrow_v[pl.ds(i * LANES, LANES)] = jnp.zeros((LANES,), jnp.float32)
    return _

  lax.fori_loop(0, rows_per_tile // LANES, initz, None)
  pltpu.sync_copy(zrow_v, shared_deg.at[pl.ds(sid * rows_per_tile,
                                              rows_per_tile)])
  plsc.subcore_barrier()

  pltpu.async_copy(dst_hbm.at[pl.ds(base, CHUNK)], didxs[0], isem)
  pltpu.async_copy(dst_hbm.at[pl.ds(base + CHUNK, CHUNK)], didxs[1], isem)

  def quad(g, _):
    for half in range(2):
      k0 = g * 4 + 2 * half
      sa = 2 * half
      sb = sa + 1
      na = (sa + 2) % 4
      nb = (na + 1) % 4
      pltpu.make_async_copy(dst_hbm.at[pl.ds(base, CHUNK)], didxs[0],
                            isem).wait()
      pltpu.make_async_copy(dst_hbm.at[pl.ds(base, CHUNK)], didxs[0],
                            isem).wait()
      dsca = pltpu.async_copy(ones_v, shared_deg.at[didxs[sa]], ssems[0],
                              add=True)
      dscb = pltpu.async_copy(ones_v, shared_deg.at[didxs[sb]], ssems[1],
                              add=True)
      pltpu.async_copy(dst_hbm.at[pl.ds(base + (k0 + 2) * CHUNK, CHUNK)],
                       didxs[na], isem)
      pltpu.async_copy(dst_hbm.at[pl.ds(base + (k0 + 3) * CHUNK, CHUNK)],
                       didxs[nb], isem)
      dsca.wait()
      dscb.wait()
    return _

  lax.fori_loop(0, chunks // 4, quad, None)
  pltpu.make_async_copy(dst_hbm.at[pl.ds(base, CHUNK)], didxs[0], isem).wait()
  pltpu.make_async_copy(dst_hbm.at[pl.ds(base, CHUNK)], didxs[0], isem).wait()
  plsc.subcore_barrier()
  n_pad = rows_per_tile * NS
  pltpu.sync_copy(shared_deg.at[pl.ds(sid * rows_per_tile, rows_per_tile)],
                  zrow_v)
  pltpu.sync_copy(
      zrow_v,
      out_hbm.at[pl.ds(cid * n_pad + sid * rows_per_tile, rows_per_tile)])


def _sc_agg_body(c_core, rows_per_tile, d_hid, src_hbm, dst_hbm, hs_hbm,
                 out_hbm, s0, s1, s2, s3, d0, d1, d2, d3, rows, gsem0, gsem1,
                 ssem0, ssem1, isem, shared_acc):
  cid = lax.axis_index("c")
  sid = lax.axis_index("s")
  per_row = d_hid // LANES
  sidxs = (s0, s1, s2, s3)
  didxs = (d0, d1, d2, d3)
  gsem = (gsem0, gsem1)
  ssem = (ssem0, ssem1)
  nz = rows_per_tile // CHUNK

  def run_core(cc, base, oc):
    def initz(i, _):
      rows[0, i // per_row, pl.ds((i % per_row) * LANES, LANES)] = (
          jnp.zeros((LANES,), jnp.float32))
      return _

    lax.fori_loop(0, CHUNK * per_row, initz, None)
    zds = []
    for j in range(nz):
      zds.append(pltpu.async_copy(
          rows.at[0],
          shared_acc.at[pl.ds(sid * rows_per_tile + j * CHUNK, CHUNK), :],
          isem))
    for d in zds:
      d.wait()
    plsc.subcore_barrier()

    pltpu.async_copy(src_hbm.at[pl.ds(base, CHUNK)], sidxs[0], isem)
    pltpu.async_copy(dst_hbm.at[pl.ds(base, CHUNK)], didxs[0], isem)
    pltpu.async_copy(src_hbm.at[pl.ds(base + CHUNK, CHUNK)], sidxs[1], isem)
    pltpu.async_copy(dst_hbm.at[pl.ds(base + CHUNK, CHUNK)], didxs[1], isem)

    def quad(g, _):
      for half in range(2):
        k0 = g * 4 + 2 * half
        sa = 2 * half
        sb = sa + 1
        na = (sa + 2) % 4
        nb = (na + 1) % 4
        for _i in range(4):
          pltpu.make_async_copy(src_hbm.at[pl.ds(base, CHUNK)], sidxs[0],
                                isem).wait()
        dga = pltpu.async_copy(hs_hbm.at[sidxs[sa]], rows.at[0], gsem[0])
        dgb = pltpu.async_copy(hs_hbm.at[sidxs[sb]], rows.at[1], gsem[1])
        offa = base + (k0 + 2) * CHUNK
        offb = base + (k0 + 3) * CHUNK
        pltpu.async_copy(src_hbm.at[pl.ds(offa, CHUNK)], sidxs[na], isem)
        pltpu.async_copy(dst_hbm.at[pl.ds(offa, CHUNK)], didxs[na], isem)
        pltpu.async_copy(src_hbm.at[pl.ds(offb, CHUNK)], sidxs[nb], isem)
        pltpu.async_copy(dst_hbm.at[pl.ds(offb, CHUNK)], didxs[nb], isem)
        dga.wait()
        dsa = pltpu.async_copy(rows.at[0], shared_acc.at[didxs[sa]], ssem[0],
                               add=True)
        dgb.wait()
        dsb = pltpu.async_copy(rows.at[1], shared_acc.at[didxs[sb]], ssem[1],
                               add=True)
        dsa.wait()
        dsb.wait()
      return _

    lax.fori_loop(0, cc // 4, quad, None)
    for _i in range(4):
      pltpu.make_async_copy(src_hbm.at[pl.ds(base, CHUNK)], sidxs[0],
                            isem).wait()
    plsc.subcore_barrier()

    outd = [None] * nz
    for j in range(nz):
      p = j % 2
      if j >= 2:
        outd[j - 2].wait()
      row0 = sid * rows_per_tile + j * CHUNK
      pltpu.async_copy(shared_acc.at[pl.ds(row0, CHUNK), :], rows.at[p],
                       gsem[p]).wait()
      outd[j] = pltpu.async_copy(rows.at[p],
                                 out_hbm.at[oc, pl.ds(row0, CHUNK), :],
                                 ssem[p])
    outd[nz - 2].wait()
    outd[nz - 1].wait()

  @pl.when(cid == 0)
  def _():
    run_core(c_core[0], (sid * c_core[0]) * CHUNK, 0)

  if c_core[1] > 0:
    @pl.when(cid == 1)
    def _():
      run_core(c_core[1], (NS * c_core[0] + sid * c_core[1]) * CHUNK, 1)


def _tc_hs_body(x_ref, w1_ref, degt_ref, hs_ref):
  deg = degt_ref[:, 0] + degt_ref[:, 1] + 1.0
  dinv = 1.0 / jnp.sqrt(deg)
  h = jnp.dot(x_ref[...], w1_ref[...], preferred_element_type=jnp.float32)
  hs_ref[...] = h * dinv[:, None]


def _tc_out_body1(a0_ref, hs_ref, degt_ref, b1_ref, w2_ref, b2_ref, out_ref):
  deg = degt_ref[:, 0] + degt_ref[:, 1] + 1.0
  dinv = 1.0 / jnp.sqrt(deg)
  t = (a0_ref[0] + hs_ref[...]) * dinv[:, None] + b1_ref[...]
  t = jnp.maximum(t, 0.0)
  out_ref[...] = jnp.dot(t, w2_ref[...],
                         preferred_element_type=jnp.float32) + b2_ref[...]


def _tc_out_body2(a0_ref, a1_ref, hs_ref, degt_ref, b1_ref, w2_ref, b2_ref,
                  out_ref):
  deg = degt_ref[:, 0] + degt_ref[:, 1] + 1.0
  dinv = 1.0 / jnp.sqrt(deg)
  t = (a0_ref[0] + a1_ref[0] + hs_ref[...]) * dinv[:, None] + b1_ref[...]
  t = jnp.maximum(t, 0.0)
  out_ref[...] = jnp.dot(t, w2_ref[...],
                         preferred_element_type=jnp.float32) + b2_ref[...]


def kernel(x, edge_index, W1, b1, W2, b2):
  n = x.shape[0]
  e = edge_index.shape[1]
  d_in = x.shape[1]
  d_hid = W1.shape[1]
  d_out = W2.shape[1]

  rows_per_tile = -(-(n + 1) // (NS * CHUNK)) * CHUNK
  n_pad = rows_per_tile * NS

  src = edge_index[0].astype(jnp.int32)
  dst = edge_index[1].astype(jnp.int32)
  tchunks = -(-e // (NW * CHUNK))
  tchunks = -(-tchunks // 4) * 4
  total = NW * tchunks
  c0 = (int(total * AGG_SPLIT) // 64) * 4
  c_core = (c0, total // NS - c0)
  pad = (total + 2) * CHUNK - e
  src_p = jnp.concatenate([src, jnp.zeros((pad,), jnp.int32)])
  pad_dst = n + jnp.arange(pad, dtype=jnp.int32) % (n_pad - n)
  dst_p = jnp.concatenate([dst, pad_dst])

  mesh = plsc.VectorSubcoreMesh(core_axis_name="c", subcore_axis_name="s")

  sc_deg = pl.kernel(
      functools.partial(_sc_deg_body, tchunks, rows_per_tile),
      out_type=jax.ShapeDtypeStruct((NC * n_pad,), jnp.float32),
      mesh=mesh,
      scratch_types=[
          pltpu.VMEM((CHUNK,), jnp.int32),
          pltpu.VMEM((CHUNK,), jnp.int32),
          pltpu.VMEM((CHUNK,), jnp.int32),
          pltpu.VMEM((CHUNK,), jnp.int32),
          pltpu.VMEM((CHUNK,), jnp.float32),
          pltpu.VMEM((rows_per_tile,), jnp.float32),
          pltpu.SemaphoreType.DMA,
          pltpu.SemaphoreType.DMA,
          pltpu.SemaphoreType.DMA,
          pltpu.VMEM_SHARED((n_pad,), jnp.float32),
      ],
  )
  degp = sc_deg(dst_p).reshape(NC, n_pad)
  degt = degp.T

  grid = n // BLK
  tc_hs = pl.pallas_call(
      _tc_hs_body,
      grid=(grid,),
      in_specs=[
          pl.BlockSpec((BLK, d_in), lambda i: (i, 0)),
          pl.BlockSpec((d_in, d_hid), lambda i: (0, 0)),
          pl.BlockSpec((BLK, NC), lambda i: (i, 0)),
      ],
      out_specs=pl.BlockSpec((BLK, d_hid), lambda i: (i, 0)),
      out_shape=jax.ShapeDtypeStruct((n, d_hid), jnp.float32),
  )
  hs = tc_hs(x, W1, degt)

  nca = 2 if c_core[1] > 0 else 1
  sc_agg = pl.kernel(
      functools.partial(_sc_agg_body, c_core, rows_per_tile, d_hid),
      out_type=jax.ShapeDtypeStruct((nca, n_pad, d_hid), jnp.float32),
      mesh=mesh,
      scratch_types=[
          pltpu.VMEM((CHUNK,), jnp.int32),
          pltpu.VMEM((CHUNK,), jnp.int32),
          pltpu.VMEM((CHUNK,), jnp.int32),
          pltpu.VMEM((CHUNK,), jnp.int32),
          pltpu.VMEM((CHUNK,), jnp.int32),
          pltpu.VMEM((CHUNK,), jnp.int32),
          pltpu.VMEM((CHUNK,), jnp.int32),
          pltpu.VMEM((CHUNK,), jnp.int32),
          pltpu.VMEM((2, CHUNK, d_hid), jnp.float32),
          pltpu.SemaphoreType.DMA,
          pltpu.SemaphoreType.DMA,
          pltpu.SemaphoreType.DMA,
          pltpu.SemaphoreType.DMA,
          pltpu.SemaphoreType.DMA,
          pltpu.VMEM_SHARED((n_pad, d_hid), jnp.float32),
      ],
  )
  aggp = sc_agg(src_p, dst_p, hs)

  a_spec = pl.BlockSpec((1, BLK, d_hid), lambda i: (0, i, 0))
  common_specs = [
      pl.BlockSpec((BLK, d_hid), lambda i: (i, 0)),
      pl.BlockSpec((BLK, NC), lambda i: (i, 0)),
      pl.BlockSpec((d_hid,), lambda i: (0,)),
      pl.BlockSpec((d_hid, d_out), lambda i: (0, 0)),
      pl.BlockSpec((d_out,), lambda i: (0,)),
  ]
  if nca == 1:
    tc_out = pl.pallas_call(
        _tc_out_body1,
        grid=(grid,),
        in_specs=[a_spec] + common_specs,
        out_specs=pl.BlockSpec((BLK, d_out), lambda i: (i, 0)),
        out_shape=jax.ShapeDtypeStruct((n, d_out), jnp.float32),
    )
    return tc_out(aggp, hs, degt, b1, W2, b2)
  tc_out = pl.pallas_call(
      _tc_out_body2,
      grid=(grid,),
      in_specs=[a_spec,
                pl.BlockSpec((1, BLK, d_hid), lambda i: (1, i, 0))]
      + common_specs,
      out_specs=pl.BlockSpec((BLK, d_out), lambda i: (i, 0)),
      out_shape=jax.ShapeDtypeStruct((n, d_out), jnp.float32),
  )
  return tc_out(aggp, aggp, hs, degt, b1, W2, b2)


# speedup vs baseline: 1.3384x; 1.3384x over previous
import functools

import jax
import jax.numpy as jnp
from jax import lax
from jax.experimental import pallas as pl
from jax.experimental.pallas import tpu as pltpu
from jax.experimental.pallas import tpu_sc as plsc

NC = 2
NS = 16
NW = NC * NS
CHUNK = 128
BLK = 1000
LANES = 16
AGG_SPLIT = 0.975


def _sc_deg_body(chunks, rows_per_tile, dst_hbm, out_hbm, d0, d1, d2, d3,
                 ones_v, zrow_v, isem, ssem0, ssem1, shared_deg):
  cid = lax.axis_index("c")
  sid = lax.axis_index("s")
  wid = cid * NS + sid
  base = wid * chunks * CHUNK
  didxs = (d0, d1, d2, d3)
  ssems = (ssem0, ssem1)

  def initc(i, _):
    ones_v[pl.ds(i * LANES, LANES)] = jnp.ones((LANES,), jnp.float32)
    return _

  lax.fori_loop(0, CHUNK // LANES, initc, None)

  def initz(i, _):
    zrow_v[pl.ds(i * LANES, LANES)] = jnp.zeros((LANES,), jnp.float32)
    return _

  lax.fori_loop(0, rows_per_tile // LANES, initz, None)
  pltpu.sync_copy(zrow_v, shared_deg.at[pl.ds(sid * rows_per_tile,
                                              rows_per_tile)])
  plsc.subcore_barrier()

  pltpu.async_copy(dst_hbm.at[pl.ds(base, CHUNK)], didxs[0], isem)
  pltpu.async_copy(dst_hbm.at[pl.ds(base + CHUNK, CHUNK)], didxs[1], isem)

  def quad(g, _):
    for half in range(2):
      k0 = g * 4 + 2 * half
      sa = 2 * half
      sb = sa + 1
      na = (sa + 2) % 4
      nb = (na + 1) % 4
      pltpu.make_async_copy(dst_hbm.at[pl.ds(base, CHUNK)], didxs[0],
                            isem).wait()
      pltpu.make_async_copy(dst_hbm.at[pl.ds(base, CHUNK)], didxs[0],
                            isem).wait()
      dsca = pltpu.async_copy(ones_v, shared_deg.at[didxs[sa]], ssems[0],
                              add=True)
      dscb = pltpu.async_copy(ones_v, shared_deg.at[didxs[sb]], ssems[1],
                              add=True)
      pltpu.async_copy(dst_hbm.at[pl.ds(base + (k0 + 2) * CHUNK, CHUNK)],
                       didxs[na], isem)
      pltpu.async_copy(dst_hbm.at[pl.ds(base + (k0 + 3) * CHUNK, CHUNK)],
                       didxs[nb], isem)
      dsca.wait()
      dscb.wait()
    return _

  lax.fori_loop(0, chunks // 4, quad, None)
  pltpu.make_async_copy(dst_hbm.at[pl.ds(base, CHUNK)], didxs[0], isem).wait()
  pltpu.make_async_copy(dst_hbm.at[pl.ds(base, CHUNK)], didxs[0], isem).wait()
  plsc.subcore_barrier()
  n_pad = rows_per_tile * NS
  pltpu.sync_copy(shared_deg.at[pl.ds(sid * rows_per_tile, rows_per_tile)],
                  zrow_v)
  pltpu.sync_copy(
      zrow_v,
      out_hbm.at[pl.ds(cid * n_pad + sid * rows_per_tile, rows_per_tile)])


def _sc_agg_body(c_core, rows_per_tile, d_hid, src_hbm, dst_hbm, hs_hbm,
                 out_hbm, s0, s1, s2, s3, d0, d1, d2, d3, rows, gsem0, gsem1,
                 ssem0, ssem1, isem, shared_acc):
  cid = lax.axis_index("c")
  sid = lax.axis_index("s")
  per_row = d_hid // LANES
  sidxs = (s0, s1, s2, s3)
  didxs = (d0, d1, d2, d3)
  gsem = (gsem0, gsem1)
  ssem = (ssem0, ssem1)
  nz = rows_per_tile // CHUNK

  def run_core(cc, base, oc):
    def initz(i, _):
      rows[0, i // per_row, pl.ds((i % per_row) * LANES, LANES)] = (
          jnp.zeros((LANES,), jnp.float32))
      return _

    lax.fori_loop(0, CHUNK * per_row, initz, None)
    zds = []
    for j in range(nz):
      zds.append(pltpu.async_copy(
          rows.at[0],
          shared_acc.at[pl.ds(sid * rows_per_tile + j * CHUNK, CHUNK), :],
          isem))
    for d in zds:
      d.wait()
    plsc.subcore_barrier()

    pltpu.async_copy(src_hbm.at[pl.ds(base, CHUNK)], sidxs[0], isem)
    pltpu.async_copy(dst_hbm.at[pl.ds(base, CHUNK)], didxs[0], isem)
    pltpu.async_copy(src_hbm.at[pl.ds(base + CHUNK, CHUNK)], sidxs[1], isem)
    pltpu.async_copy(dst_hbm.at[pl.ds(base + CHUNK, CHUNK)], didxs[1], isem)

    def quad(g, _):
      for half in range(2):
        k0 = g * 4 + 2 * half
        sa = 2 * half
        sb = sa + 1
        na = (sa + 2) % 4
        nb = (na + 1) % 4
        for _i in range(4):
          pltpu.make_async_copy(src_hbm.at[pl.ds(base, CHUNK)], sidxs[0],
                                isem).wait()
        dga = pltpu.async_copy(hs_hbm.at[sidxs[sa]], rows.at[0], gsem[0])
        dgb = pltpu.async_copy(hs_hbm.at[sidxs[sb]], rows.at[1], gsem[1])
        offa = base + (k0 + 2) * CHUNK
        offb = base + (k0 + 3) * CHUNK
        pltpu.async_copy(src_hbm.at[pl.ds(offa, CHUNK)], sidxs[na], isem)
        pltpu.async_copy(dst_hbm.at[pl.ds(offa, CHUNK)], didxs[na], isem)
        pltpu.async_copy(src_hbm.at[pl.ds(offb, CHUNK)], sidxs[nb], isem)
        pltpu.async_copy(dst_hbm.at[pl.ds(offb, CHUNK)], didxs[nb], isem)
        dga.wait()
        dsa = pltpu.async_copy(rows.at[0], shared_acc.at[didxs[sa]], ssem[0],
                               add=True)
        dgb.wait()
        dsb = pltpu.async_copy(rows.at[1], shared_acc.at[didxs[sb]], ssem[1],
                               add=True)
        dsa.wait()
        dsb.wait()
      return _

    lax.fori_loop(0, cc // 4, quad, None)
    for _i in range(4):
      pltpu.make_async_copy(src_hbm.at[pl.ds(base, CHUNK)], sidxs[0],
                            isem).wait()
    plsc.subcore_barrier()

    outd = [None] * nz
    for j in range(nz):
      p = j % 2
      if j >= 2:
        outd[j - 2].wait()
      row0 = sid * rows_per_tile + j * CHUNK
      pltpu.async_copy(shared_acc.at[pl.ds(row0, CHUNK), :], rows.at[p],
                       gsem[p]).wait()
      outd[j] = pltpu.async_copy(rows.at[p],
                                 out_hbm.at[oc, pl.ds(row0, CHUNK), :],
                                 ssem[p])
    outd[nz - 2].wait()
    outd[nz - 1].wait()

  @pl.when(cid == 0)
  def _():
    run_core(c_core[0], (sid * c_core[0]) * CHUNK, 0)

  if c_core[1] > 0:
    @pl.when(cid == 1)
    def _():
      run_core(c_core[1], (NS * c_core[0] + sid * c_core[1]) * CHUNK, 1)


def _tc_hs_body(x_ref, w1_ref, degt_ref, hs_ref):
  deg = degt_ref[:, 0] + degt_ref[:, 1] + 1.0
  dinv = 1.0 / jnp.sqrt(deg)
  h = jnp.dot(x_ref[...], w1_ref[...], preferred_element_type=jnp.float32)
  hs_ref[...] = h * dinv[:, None]


def _tc_out_body1(a0_ref, hs_ref, degt_ref, b1_ref, w2_ref, b2_ref, out_ref):
  deg = degt_ref[:, 0] + degt_ref[:, 1] + 1.0
  dinv = 1.0 / jnp.sqrt(deg)
  t = (a0_ref[0] + hs_ref[...]) * dinv[:, None] + b1_ref[...]
  t = jnp.maximum(t, 0.0)
  out_ref[...] = jnp.dot(t, w2_ref[...],
                         preferred_element_type=jnp.float32) + b2_ref[...]


def _tc_out_body2(a0_ref, a1_ref, hs_ref, degt_ref, b1_ref, w2_ref, b2_ref,
                  out_ref):
  deg = degt_ref[:, 0] + degt_ref[:, 1] + 1.0
  dinv = 1.0 / jnp.sqrt(deg)
  t = (a0_ref[0] + a1_ref[0] + hs_ref[...]) * dinv[:, None] + b1_ref[...]
  t = jnp.maximum(t, 0.0)
  out_ref[...] = jnp.dot(t, w2_ref[...],
                         preferred_element_type=jnp.float32) + b2_ref[...]


def kernel(x, edge_index, W1, b1, W2, b2):
  n = x.shape[0]
  e = edge_index.shape[1]
  d_in = x.shape[1]
  d_hid = W1.shape[1]
  d_out = W2.shape[1]

  rows_per_tile = -(-(n + 1) // (NS * CHUNK)) * CHUNK
  n_pad = rows_per_tile * NS

  src = edge_index[0].astype(jnp.int32)
  dst = edge_index[1].astype(jnp.int32)
  tchunks = -(-e // (NW * CHUNK))
  tchunks = -(-tchunks // 4) * 4
  total = NW * tchunks
  c0 = (int(total * AGG_SPLIT) // 64) * 4
  c_core = (c0, total // NS - c0)
  pad = (total + 2) * CHUNK - e
  src_p = jnp.concatenate([src, jnp.zeros((pad,), jnp.int32)])
  pad_dst = n + jnp.arange(pad, dtype=jnp.int32) % (n_pad - n)
  dst_p = jnp.concatenate([dst, pad_dst])

  mesh = plsc.VectorSubcoreMesh(core_axis_name="c", subcore_axis_name="s")

  sc_deg = pl.kernel(
      functools.partial(_sc_deg_body, tchunks, rows_per_tile),
      out_type=jax.ShapeDtypeStruct((NC * n_pad,), jnp.float32),
      mesh=mesh,
      scratch_types=[
          pltpu.VMEM((CHUNK,), jnp.int32),
          pltpu.VMEM((CHUNK,), jnp.int32),
          pltpu.VMEM((CHUNK,), jnp.int32),
          pltpu.VMEM((CHUNK,), jnp.int32),
          pltpu.VMEM((CHUNK,), jnp.float32),
          pltpu.VMEM((rows_per_tile,), jnp.float32),
          pltpu.SemaphoreType.DMA,
          pltpu.SemaphoreType.DMA,
          pltpu.SemaphoreType.DMA,
          pltpu.VMEM_SHARED((n_pad,), jnp.float32),
      ],
  )
  degp = sc_deg(dst_p).reshape(NC, n_pad)
  degt = degp.T

  grid = n // BLK
  tc_hs = pl.pallas_call(
      _tc_hs_body,
      grid=(grid,),
      in_specs=[
          pl.BlockSpec((BLK, d_in), lambda i: (i, 0)),
          pl.BlockSpec((d_in, d_hid), lambda i: (0, 0)),
          pl.BlockSpec((BLK, NC), lambda i: (i, 0)),
      ],
      out_specs=pl.BlockSpec((BLK, d_hid), lambda i: (i, 0)),
      out_shape=jax.ShapeDtypeStruct((n, d_hid), jnp.float32),
  )
  hs = tc_hs(x, W1, degt)

  nca = 2 if c_core[1] > 0 else 1
  sc_agg = pl.kernel(
      functools.partial(_sc_agg_body, c_core, rows_per_tile, d_hid),
      out_type=jax.ShapeDtypeStruct((nca, n_pad, d_hid), jnp.float32),
      mesh=mesh,
      scratch_types=[
          pltpu.VMEM((CHUNK,), jnp.int32),
          pltpu.VMEM((CHUNK,), jnp.int32),
          pltpu.VMEM((CHUNK,), jnp.int32),
          pltpu.VMEM((CHUNK,), jnp.int32),
          pltpu.VMEM((CHUNK,), jnp.int32),
          pltpu.VMEM((CHUNK,), jnp.int32),
          pltpu.VMEM((CHUNK,), jnp.int32),
          pltpu.VMEM((CHUNK,), jnp.int32),
          pltpu.VMEM((2, CHUNK, d_hid), jnp.float32),
          pltpu.SemaphoreType.DMA,
          pltpu.SemaphoreType.DMA,
          pltpu.SemaphoreType.DMA,
          pltpu.SemaphoreType.DMA,
          pltpu.SemaphoreType.DMA,
          pltpu.VMEM_SHARED((n_pad, d_hid), jnp.float32),
      ],
  )
  aggp = sc_agg(src_p, dst_p, hs)

  a_spec = pl.BlockSpec((1, BLK, d_hid), lambda i: (0, i, 0))
  common_specs = [
      pl.BlockSpec((BLK, d_hid), lambda i: (i, 0)),
      pl.BlockSpec((BLK, NC), lambda i: (i, 0)),
      pl.BlockSpec((d_hid,), lambda i: (0,)),
      pl.BlockSpec((d_hid, d_out), lambda i: (0, 0)),
      pl.BlockSpec((d_out,), lambda i: (0,)),
  ]
  if nca == 1:
    tc_out = pl.pallas_call(
        _tc_out_body1,
        grid=(grid,),
        in_specs=[a_spec] + common_specs,
        out_specs=pl.BlockSpec((BLK, d_out), lambda i: (i, 0)),
        out_shape=jax.ShapeDtypeStruct((n, d_out), jnp.float32),
    )
    return tc_out(aggp, hs, degt, b1, W2, b2)
  tc_out = pl.pallas_call(
      _tc_out_body2,
      grid=(grid,),
      in_specs=[a_spec,
                pl.BlockSpec((1, BLK, d_hid), lambda i: (1, i, 0))]
      + common_specs,
      out_specs=pl.BlockSpec((BLK, d_out), lambda i: (i, 0)),
      out_shape=jax.ShapeDtypeStruct((n, d_out), jnp.float32),
  )
  return tc_out(aggp, aggp, hs, degt, b1, W2, b2)


# speedup vs baseline: 1.4230x; 1.0632x over previous
import functools

import jax
import jax.numpy as jnp
from jax import lax
from jax.experimental import pallas as pl
from jax.experimental.pallas import tpu as pltpu
from jax.experimental.pallas import tpu_sc as plsc

NC = 2
NS = 16
NW = NC * NS
CHUNK = 128
BLK = 1000
LANES = 16
AGG_SPLIT = 0.85


def _sc_deg_body(chunks, rows_per_tile, dst_hbm, out_hbm, d0, d1, d2, d3,
                 ones_v, zrow_v, isem, ssem0, ssem1, shared_deg):
  cid = lax.axis_index("c")
  sid = lax.axis_index("s")
  wid = cid * NS + sid
  base = wid * chunks * CHUNK
  didxs = (d0, d1, d2, d3)
  ssems = (ssem0, ssem1)

  def initc(i, _):
    ones_v[pl.ds(i * LANES, LANES)] = jnp.ones((LANES,), jnp.float32)
    return _

  lax.fori_loop(0, CHUNK // LANES, initc, None)

  def initz(i, _):
    zrow_v[pl.ds(i * LANES, LANES)] = jnp.zeros((LANES,), jnp.float32)
    return _

  lax.fori_loop(0, rows_per_tile // LANES, initz, None)
  pltpu.sync_copy(zrow_v, shared_deg.at[pl.ds(sid * rows_per_tile,
                                              rows_per_tile)])
  plsc.subcore_barrier()

  pltpu.async_copy(dst_hbm.at[pl.ds(base, CHUNK)], didxs[0], isem)
  pltpu.async_copy(dst_hbm.at[pl.ds(base + CHUNK, CHUNK)], didxs[1], isem)

  def quad(g, _):
    for half in range(2):
      k0 = g * 4 + 2 * half
      sa = 2 * half
      sb = sa + 1
      na = (sa + 2) % 4
      nb = (na + 1) % 4
      pltpu.make_async_copy(dst_hbm.at[pl.ds(base, CHUNK)], didxs[0],
                            isem).wait()
      pltpu.make_async_copy(dst_hbm.at[pl.ds(base, CHUNK)], didxs[0],
                            isem).wait()
      dsca = pltpu.async_copy(ones_v, shared_deg.at[didxs[sa]], ssems[0],
                              add=True)
      dscb = pltpu.async_copy(ones_v, shared_deg.at[didxs[sb]], ssems[1],
                              add=True)
      pltpu.async_copy(dst_hbm.at[pl.ds(base + (k0 + 2) * CHUNK, CHUNK)],
                       didxs[na], isem)
      pltpu.async_copy(dst_hbm.at[pl.ds(base + (k0 + 3) * CHUNK, CHUNK)],
                       didxs[nb], isem)
      dsca.wait()
      dscb.wait()
    return _

  lax.fori_loop(0, chunks // 4, quad, None)
  pltpu.make_async_copy(dst_hbm.at[pl.ds(base, CHUNK)], didxs[0], isem).wait()
  pltpu.make_async_copy(dst_hbm.at[pl.ds(base, CHUNK)], didxs[0], isem).wait()
  plsc.subcore_barrier()
  n_pad = rows_per_tile * NS
  pltpu.sync_copy(shared_deg.at[pl.ds(sid * rows_per_tile, rows_per_tile)],
                  zrow_v)
  pltpu.sync_copy(
      zrow_v,
      out_hbm.at[pl.ds(cid * n_pad + sid * rows_per_tile, rows_per_tile)])


def _sc_agg_body(c_core, rows_per_tile, d_hid, src_hbm, dst_hbm, hs_hbm,
                 out_hbm, s0, s1, s2, s3, d0, d1, d2, d3, rows, gsem0, gsem1,
                 ssem0, ssem1, isem, shared_acc):
  cid = lax.axis_index("c")
  sid = lax.axis_index("s")
  per_row = d_hid // LANES
  sidxs = (s0, s1, s2, s3)
  didxs = (d0, d1, d2, d3)
  gsem = (gsem0, gsem1)
  ssem = (ssem0, ssem1)
  nz = rows_per_tile // CHUNK

  def run_core(cc, base, oc):
    def initz(i, _):
      rows[0, i // per_row, pl.ds((i % per_row) * LANES, LANES)] = (
          jnp.zeros((LANES,), jnp.float32))
      return _

    lax.fori_loop(0, CHUNK * per_row, initz, None)
    zds = []
    for j in range(nz):
      zds.append(pltpu.async_copy(
          rows.at[0],
          shared_acc.at[pl.ds(sid * rows_per_tile + j * CHUNK, CHUNK), :],
          isem))
    for d in zds:
      d.wait()
    plsc.subcore_barrier()

    pltpu.async_copy(src_hbm.at[pl.ds(base, CHUNK)], sidxs[0], isem)
    pltpu.async_copy(dst_hbm.at[pl.ds(base, CHUNK)], didxs[0], isem)
    pltpu.async_copy(src_hbm.at[pl.ds(base + CHUNK, CHUNK)], sidxs[1], isem)
    pltpu.async_copy(dst_hbm.at[pl.ds(base + CHUNK, CHUNK)], didxs[1], isem)

    def quad(g, _):
      for half in range(2):
        k0 = g * 4 + 2 * half
        sa = 2 * half
        sb = sa + 1
        na = (sa + 2) % 4
        nb = (na + 1) % 4
        for _i in range(4):
          pltpu.make_async_copy(src_hbm.at[pl.ds(base, CHUNK)], sidxs[0],
                                isem).wait()
        dga = pltpu.async_copy(hs_hbm.at[sidxs[sa]], rows.at[0], gsem[0])
        dgb = pltpu.async_copy(hs_hbm.at[sidxs[sb]], rows.at[1], gsem[1])
        offa = base + (k0 + 2) * CHUNK
        offb = base + (k0 + 3) * CHUNK
        pltpu.async_copy(src_hbm.at[pl.ds(offa, CHUNK)], sidxs[na], isem)
        pltpu.async_copy(dst_hbm.at[pl.ds(offa, CHUNK)], didxs[na], isem)
        pltpu.async_copy(src_hbm.at[pl.ds(offb, CHUNK)], sidxs[nb], isem)
        pltpu.async_copy(dst_hbm.at[pl.ds(offb, CHUNK)], didxs[nb], isem)
        dga.wait()
        dsa = pltpu.async_copy(rows.at[0], shared_acc.at[didxs[sa]], ssem[0],
                               add=True)
        dgb.wait()
        dsb = pltpu.async_copy(rows.at[1], shared_acc.at[didxs[sb]], ssem[1],
                               add=True)
        dsa.wait()
        dsb.wait()
      return _

    lax.fori_loop(0, cc // 4, quad, None)
    for _i in range(4):
      pltpu.make_async_copy(src_hbm.at[pl.ds(base, CHUNK)], sidxs[0],
                            isem).wait()
    plsc.subcore_barrier()

    outd = [None] * nz
    for j in range(nz):
      p = j % 2
      if j >= 2:
        outd[j - 2].wait()
      row0 = sid * rows_per_tile + j * CHUNK
      pltpu.async_copy(shared_acc.at[pl.ds(row0, CHUNK), :], rows.at[p],
                       gsem[p]).wait()
      outd[j] = pltpu.async_copy(rows.at[p],
                                 out_hbm.at[oc, pl.ds(row0, CHUNK), :],
                                 ssem[p])
    outd[nz - 2].wait()
    outd[nz - 1].wait()

  @pl.when(cid == 0)
  def _():
    run_core(c_core[0], (sid * c_core[0]) * CHUNK, 0)

  if c_core[1] > 0:
    @pl.when(cid == 1)
    def _():
      run_core(c_core[1], (NS * c_core[0] + sid * c_core[1]) * CHUNK, 1)


def _tc_hs_body(x_ref, w1_ref, degt_ref, hs_ref):
  deg = degt_ref[:, 0] + degt_ref[:, 1] + 1.0
  dinv = 1.0 / jnp.sqrt(deg)
  h = jnp.dot(x_ref[...], w1_ref[...], preferred_element_type=jnp.float32)
  hs_ref[...] = h * dinv[:, None]


def _tc_out_body1(a0_ref, hs_ref, degt_ref, b1_ref, w2_ref, b2_ref, out_ref):
  deg = degt_ref[:, 0] + degt_ref[:, 1] + 1.0
  dinv = 1.0 / jnp.sqrt(deg)
  t = (a0_ref[0] + hs_ref[...]) * dinv[:, None] + b1_ref[...]
  t = jnp.maximum(t, 0.0)
  out_ref[...] = jnp.dot(t, w2_ref[...],
                         preferred_element_type=jnp.float32) + b2_ref[...]


def _tc_out_body2(a0_ref, a1_ref, hs_ref, degt_ref, b1_ref, w2_ref, b2_ref,
                  out_ref):
  deg = degt_ref[:, 0] + degt_ref[:, 1] + 1.0
  dinv = 1.0 / jnp.sqrt(deg)
  t = (a0_ref[0] + a1_ref[0] + hs_ref[...]) * dinv[:, None] + b1_ref[...]
  t = jnp.maximum(t, 0.0)
  out_ref[...] = jnp.dot(t, w2_ref[...],
                         preferred_element_type=jnp.float32) + b2_ref[...]


def kernel(x, edge_index, W1, b1, W2, b2):
  n = x.shape[0]
  e = edge_index.shape[1]
  d_in = x.shape[1]
  d_hid = W1.shape[1]
  d_out = W2.shape[1]

  rows_per_tile = -(-(n + 1) // (NS * CHUNK)) * CHUNK
  n_pad = rows_per_tile * NS

  src = edge_index[0].astype(jnp.int32)
  dst = edge_index[1].astype(jnp.int32)
  tchunks = -(-e // (NW * CHUNK))
  tchunks = -(-tchunks // 4) * 4
  total = NW * tchunks
  c0 = (int(total * AGG_SPLIT) // 64) * 4
  c_core = (c0, total // NS - c0)
  pad = (total + 2) * CHUNK - e
  src_p = jnp.concatenate([src, jnp.zeros((pad,), jnp.int32)])
  pad_dst = n + jnp.arange(pad, dtype=jnp.int32) % (n_pad - n)
  dst_p = jnp.concatenate([dst, pad_dst])

  mesh = plsc.VectorSubcoreMesh(core_axis_name="c", subcore_axis_name="s")

  sc_deg = pl.kernel(
      functools.partial(_sc_deg_body, tchunks, rows_per_tile),
      out_type=jax.ShapeDtypeStruct((NC * n_pad,), jnp.float32),
      mesh=mesh,
      scratch_types=[
          pltpu.VMEM((CHUNK,), jnp.int32),
          pltpu.VMEM((CHUNK,), jnp.int32),
          pltpu.VMEM((CHUNK,), jnp.int32),
          pltpu.VMEM((CHUNK,), jnp.int32),
          pltpu.VMEM((CHUNK,), jnp.float32),
          pltpu.VMEM((rows_per_tile,), jnp.float32),
          pltpu.SemaphoreType.DMA,
          pltpu.SemaphoreType.DMA,
          pltpu.SemaphoreType.DMA,
          pltpu.VMEM_SHARED((n_pad,), jnp.float32),
      ],
  )
  degp = sc_deg(dst_p).reshape(NC, n_pad)
  degt = degp.T

  grid = n // BLK
  tc_hs = pl.pallas_call(
      _tc_hs_body,
      grid=(grid,),
      in_specs=[
          pl.BlockSpec((BLK, d_in), lambda i: (i, 0)),
          pl.BlockSpec((d_in, d_hid), lambda i: (0, 0)),
          pl.BlockSpec((BLK, NC), lambda i: (i, 0)),
      ],
      out_specs=pl.BlockSpec((BLK, d_hid), lambda i: (i, 0)),
      out_shape=jax.ShapeDtypeStruct((n, d_hid), jnp.float32),
  )
  hs = tc_hs(x, W1, degt)

  nca = 2 if c_core[1] > 0 else 1
  sc_agg = pl.kernel(
      functools.partial(_sc_agg_body, c_core, rows_per_tile, d_hid),
      out_type=jax.ShapeDtypeStruct((nca, n_pad, d_hid), jnp.float32),
      mesh=mesh,
      scratch_types=[
          pltpu.VMEM((CHUNK,), jnp.int32),
          pltpu.VMEM((CHUNK,), jnp.int32),
          pltpu.VMEM((CHUNK,), jnp.int32),
          pltpu.VMEM((CHUNK,), jnp.int32),
          pltpu.VMEM((CHUNK,), jnp.int32),
          pltpu.VMEM((CHUNK,), jnp.int32),
          pltpu.VMEM((CHUNK,), jnp.int32),
          pltpu.VMEM((CHUNK,), jnp.int32),
          pltpu.VMEM((2, CHUNK, d_hid), jnp.float32),
          pltpu.SemaphoreType.DMA,
          pltpu.SemaphoreType.DMA,
          pltpu.SemaphoreType.DMA,
          pltpu.SemaphoreType.DMA,
          pltpu.SemaphoreType.DMA,
          pltpu.VMEM_SHARED((n_pad, d_hid), jnp.float32),
      ],
  )
  aggp = sc_agg(src_p, dst_p, hs)

  a_spec = pl.BlockSpec((1, BLK, d_hid), lambda i: (0, i, 0))
  common_specs = [
      pl.BlockSpec((BLK, d_hid), lambda i: (i, 0)),
      pl.BlockSpec((BLK, NC), lambda i: (i, 0)),
      pl.BlockSpec((d_hid,), lambda i: (0,)),
      pl.BlockSpec((d_hid, d_out), lambda i: (0, 0)),
      pl.BlockSpec((d_out,), lambda i: (0,)),
  ]
  if nca == 1:
    tc_out = pl.pallas_call(
        _tc_out_body1,
        grid=(grid,),
        in_specs=[a_spec] + common_specs,
        out_specs=pl.BlockSpec((BLK, d_out), lambda i: (i, 0)),
        out_shape=jax.ShapeDtypeStruct((n, d_out), jnp.float32),
    )
    return tc_out(aggp, hs, degt, b1, W2, b2)
  tc_out = pl.pallas_call(
      _tc_out_body2,
      grid=(grid,),
      in_specs=[a_spec,
                pl.BlockSpec((1, BLK, d_hid), lambda i: (1, i, 0))]
      + common_specs,
      out_specs=pl.BlockSpec((BLK, d_out), lambda i: (i, 0)),
      out_shape=jax.ShapeDtypeStruct((n, d_out), jnp.float32),
  )
  return tc_out(aggp, aggp, hs, degt, b1, W2, b2)
